# Initial kernel scaffold; baseline (speedup 1.0000x reference)
import jax
import jax.numpy as jnp
from jax.experimental import pallas as pl


def kernel(x, edge_index, laplacian_index, laplacian_weight, W1, b1, W2, b2, W3, b3, W4, b4):
    raise NotImplementedError("write your pallas kernel here")


# speedup vs baseline: 24.6368x; 24.6368x over previous
import functools

import jax
import jax.numpy as jnp
from jax import lax
from jax.experimental import pallas as pl
from jax.experimental.pallas import tpu as pltpu
from jax.experimental.pallas import tpu_sc as plsc

N = 10000
E = 320000
D = 128

NC, NS, L = 2, 16, 16
NW = NC * NS
CH = 128
CPT = 80
EPT = CH * CPT
EPAD = EPT * NW
NPAD = 10240
RPT = NPAD // NS
DUMMY = NPAD - 1


def _sc_mesh():
    return plsc.VectorSubcoreMesh(core_axis_name="c", subcore_axis_name="s",
                                  num_cores=NC, num_subcores=NS)


def _deg_body(dst_hbm, zeros_hbm, out_hbm, dst_v, ones_v, acc_sh):
    c = lax.axis_index("c")
    s = lax.axis_index("s")
    wid = c * NS + s
    lo = s * RPT
    pltpu.sync_copy(zeros_hbm.at[pl.ds(lo, RPT)], acc_sh.at[pl.ds(lo, RPT)])
    pltpu.sync_copy(dst_hbm.at[wid], dst_v)
    for i in range(CH // L):
        ones_v[pl.ds(i * L, L)] = jnp.full((L,), 1.0, jnp.float32)
    plsc.subcore_barrier()

    def chunk(j, carry):
        pltpu.sync_copy(ones_v, acc_sh.at[dst_v.at[j]], add=True)
        return carry

    lax.fori_loop(0, CPT, chunk, 0)
    plsc.subcore_barrier()
    pltpu.sync_copy(acc_sh.at[pl.ds(lo, RPT)],
                    out_hbm.at[pl.ds(c * NPAD + lo, RPT)])


_SC_PARAMS = pltpu.CompilerParams(use_tc_tiling_on_sc=False)

_deg_kernel = functools.partial(
    pl.kernel,
    out_type=jax.ShapeDtypeStruct((NC * NPAD,), jnp.float32),
    mesh=_sc_mesh(),
    compiler_params=_SC_PARAMS,
    scratch_types=[
        pltpu.VMEM((CPT, CH), jnp.int32),
        pltpu.VMEM((CH,), jnp.float32),
        pltpu.VMEM_SHARED((NPAD,), jnp.float32),
    ],
)(_deg_body)


def _prop_body(g_hbm, src_hbm, dst_hbm, zeros_hbm, out_hbm,
               src_v, dst_v, rows_v, acc_sh, sem):
    c = lax.axis_index("c")
    s = lax.axis_index("s")
    wid = c * NS + s
    lo = s * RPT
    pltpu.sync_copy(zeros_hbm.at[pl.ds(lo, RPT)], acc_sh.at[pl.ds(lo, RPT)])
    pltpu.sync_copy(src_hbm.at[wid], src_v)
    pltpu.sync_copy(dst_hbm.at[wid], dst_v)
    plsc.subcore_barrier()

    def chunk(j, carry):
        pltpu.async_copy(g_hbm.at[src_v.at[j]], rows_v, sem).wait()
        pltpu.sync_copy(rows_v, acc_sh.at[dst_v.at[j]], add=True)
        return carry

    lax.fori_loop(0, CPT, chunk, 0)
    plsc.subcore_barrier()
    pltpu.sync_copy(acc_sh.at[pl.ds(lo, RPT)],
                    out_hbm.at[pl.ds(c * NPAD + lo, RPT)])


def _make_prop(F):
    return functools.partial(
        pl.kernel,
        out_type=jax.ShapeDtypeStruct((NC * NPAD, F), jnp.float32),
        mesh=_sc_mesh(),
        compiler_params=_SC_PARAMS,
        scratch_types=[
            pltpu.VMEM((CPT, CH), jnp.int32),
            pltpu.VMEM((CPT, CH), jnp.int32),
            pltpu.VMEM((CH, F), jnp.float32),
            pltpu.VMEM_SHARED((NPAD, F), jnp.float32),
            pltpu.SemaphoreType.DMA,
        ],
    )(_prop_body)


_prop = {F: _make_prop(F) for F in (8, 16)}


BLK = 2000


def _t1_body(dg0_ref, dg1_ref, x_ref, w_ref, dinv_ref, g_ref):
    deg = dg0_ref[...] + dg1_ref[...] + 1.0
    dinv = 1.0 / jnp.sqrt(deg)
    dinv_ref[...] = dinv
    h = jnp.dot(x_ref[...], w_ref[...], preferred_element_type=jnp.float32)
    g_ref[...] = dinv * h


def _t1(dg0, dg1, x, W1):
    return pl.pallas_call(
        _t1_body,
        grid=(N // BLK,),
        in_specs=[
            pl.BlockSpec((BLK, 1), lambda i: (i, 0)),
            pl.BlockSpec((BLK, 1), lambda i: (i, 0)),
            pl.BlockSpec((BLK, D), lambda i: (i, 0)),
            pl.BlockSpec((D, 8), lambda i: (0, 0)),
        ],
        out_specs=[
            pl.BlockSpec((BLK, 1), lambda i: (i, 0)),
            pl.BlockSpec((BLK, 8), lambda i: (i, 0)),
        ],
        out_shape=[
            jax.ShapeDtypeStruct((N, 1), jnp.float32),
            jax.ShapeDtypeStruct((N, 8), jnp.float32),
        ],
    )(dg0, dg1, x, W1)


def _tmid_body(a0_ref, a1_ref, g_ref, dinv_ref, b_ref, w_ref, o_ref):
    dinv = dinv_ref[...]
    act = dinv * (a0_ref[...] + a1_ref[...] + g_ref[...]) + b_ref[...]
    act = jnp.where(act > 0, act, jnp.exp(act) - 1.0)
    h = jnp.dot(act, w_ref[...], preferred_element_type=jnp.float32)
    o_ref[...] = dinv * h


def _tmid(a0, a1, g, dinv, b, W):
    Fi, Fo = W.shape
    return pl.pallas_call(
        _tmid_body,
        grid=(N // BLK,),
        in_specs=[
            pl.BlockSpec((BLK, Fi), lambda i: (i, 0)),
            pl.BlockSpec((BLK, Fi), lambda i: (i, 0)),
            pl.BlockSpec((BLK, Fi), lambda i: (i, 0)),
            pl.BlockSpec((BLK, 1), lambda i: (i, 0)),
            pl.BlockSpec((1, Fi), lambda i: (0, 0)),
            pl.BlockSpec((Fi, Fo), lambda i: (0, 0)),
        ],
        out_specs=pl.BlockSpec((BLK, Fo), lambda i: (i, 0)),
        out_shape=jax.ShapeDtypeStruct((N, Fo), jnp.float32),
    )(a0, a1, g, dinv, b, W)


def _tfinal_body(a0_ref, a1_ref, g_ref, dinv_ref, b_ref, o_ref):
    act = dinv_ref[...] * (a0_ref[...] + a1_ref[...] + g_ref[...]) + b_ref[...]
    m = jnp.max(act, axis=1, keepdims=True)
    sft = act - m
    o_ref[...] = sft - jnp.log(jnp.sum(jnp.exp(sft), axis=1, keepdims=True))


def _tfinal(a0, a1, g, dinv, b):
    F = g.shape[1]
    return pl.pallas_call(
        _tfinal_body,
        grid=(N // BLK,),
        in_specs=[
            pl.BlockSpec((BLK, F), lambda i: (i, 0)),
            pl.BlockSpec((BLK, F), lambda i: (i, 0)),
            pl.BlockSpec((BLK, F), lambda i: (i, 0)),
            pl.BlockSpec((BLK, 1), lambda i: (i, 0)),
            pl.BlockSpec((1, F), lambda i: (0, 0)),
        ],
        out_specs=pl.BlockSpec((BLK, F), lambda i: (i, 0)),
        out_shape=jax.ShapeDtypeStruct((N, F), jnp.float32),
    )(a0, a1, g, dinv, b)


def kernel(x, edge_index, laplacian_index, laplacian_weight,
           W1, b1, W2, b2, W3, b3, W4, b4):
    del laplacian_index, laplacian_weight
    src = edge_index[0]
    dst = edge_index[1]
    npad = EPAD - E
    src_r = jnp.concatenate([src, jnp.zeros((npad,), jnp.int32)]
                            ).reshape(NW, CPT, CH)
    dst_r = jnp.concatenate([dst, jnp.full((npad,), DUMMY, jnp.int32)]
                            ).reshape(NW, CPT, CH)

    zeros1 = jnp.zeros((NPAD,), jnp.float32)
    deg2 = _deg_kernel(dst_r, zeros1)
    dg0 = deg2[:N, None]
    dg1 = deg2[NPAD:NPAD + N, None]

    dinv, g = _t1(dg0, dg1, x, W1)

    for (b, Wn) in ((b1, W2), (b2, W3), (b3, W4)):
        F = g.shape[1]
        acc = _prop[F](g, src_r, dst_r, jnp.zeros((NPAD, F), jnp.float32))
        g = _tmid(acc[:N], acc[NPAD:NPAD + N], g, dinv,
                  b.reshape(1, F), Wn)

    F = g.shape[1]
    gp = jnp.pad(g, ((0, 0), (0, 8 - F)))
    acc = _prop[8](gp, src_r, dst_r, jnp.zeros((NPAD, 8), jnp.float32))
    return _tfinal(acc[:N, :F], acc[NPAD:NPAD + N, :F], g, dinv,
                   b4.reshape(1, F))


# speedup vs baseline: 36.2274x; 1.4705x over previous
import functools

import jax
import jax.numpy as jnp
from jax import lax
from jax.experimental import pallas as pl
from jax.experimental.pallas import tpu as pltpu
from jax.experimental.pallas import tpu_sc as plsc

N = 10000
E = 320000
D = 128

NC, NS, L = 2, 16, 16
NW = NC * NS
CH = 128
CPT = 80
EPT = CH * CPT
EPAD = EPT * NW
NPAD = 10240
RPT = NPAD // NS
DUMMY = NPAD - 1


def _sc_mesh():
    return plsc.VectorSubcoreMesh(core_axis_name="c", subcore_axis_name="s",
                                  num_cores=NC, num_subcores=NS)


def _deg_body(dst_hbm, zeros_hbm, out_hbm, dst_v, ones_v, acc_sh):
    c = lax.axis_index("c")
    s = lax.axis_index("s")
    wid = c * NS + s
    lo = s * RPT
    pltpu.sync_copy(zeros_hbm.at[pl.ds(lo, RPT)], acc_sh.at[pl.ds(lo, RPT)])
    pltpu.sync_copy(dst_hbm.at[wid], dst_v)
    for i in range(CH // L):
        ones_v[pl.ds(i * L, L)] = jnp.full((L,), 1.0, jnp.float32)
    plsc.subcore_barrier()

    def chunk(j, carry):
        pltpu.sync_copy(ones_v, acc_sh.at[dst_v.at[j]], add=True)
        return carry

    lax.fori_loop(0, CPT, chunk, 0)
    plsc.subcore_barrier()
    pltpu.sync_copy(acc_sh.at[pl.ds(lo, RPT)],
                    out_hbm.at[pl.ds(c * NPAD + lo, RPT)])


_SC_PARAMS = pltpu.CompilerParams(use_tc_tiling_on_sc=False)

_deg_kernel = functools.partial(
    pl.kernel,
    out_type=jax.ShapeDtypeStruct((NC * NPAD,), jnp.float32),
    mesh=_sc_mesh(),
    compiler_params=_SC_PARAMS,
    scratch_types=[
        pltpu.VMEM((CPT, CH), jnp.int32),
        pltpu.VMEM((CH,), jnp.float32),
        pltpu.VMEM_SHARED((NPAD,), jnp.float32),
    ],
)(_deg_body)


NBUF = 4


def _prop_body(g_hbm, src_hbm, dst_hbm, zeros_hbm, out_hbm,
               src_v, dst_v, rows_v, acc_sh, *sems):
    c = lax.axis_index("c")
    s = lax.axis_index("s")
    wid = c * NS + s
    lo = s * RPT
    pltpu.async_copy(zeros_hbm.at[pl.ds(lo, RPT)], acc_sh.at[pl.ds(lo, RPT)],
                     sems[0])
    pltpu.async_copy(src_hbm.at[wid], src_v, sems[1])
    pltpu.async_copy(dst_hbm.at[wid], dst_v, sems[2])
    pltpu.make_async_copy(zeros_hbm.at[pl.ds(lo, RPT)],
                          acc_sh.at[pl.ds(lo, RPT)], sems[0]).wait()
    pltpu.make_async_copy(src_hbm.at[wid], src_v, sems[1]).wait()
    pltpu.make_async_copy(dst_hbm.at[wid], dst_v, sems[2]).wait()
    plsc.subcore_barrier()

    for b in range(NBUF):
        pltpu.async_copy(g_hbm.at[src_v.at[b]], rows_v.at[b], sems[b])

    def group(i, carry):
        base = i * NBUF
        for b in range(NBUF):
            cch = base + b
            pltpu.make_async_copy(g_hbm.at[src_v.at[cch]], rows_v.at[b],
                                  sems[b]).wait()
            pltpu.sync_copy(rows_v.at[b], acc_sh.at[dst_v.at[cch]], add=True)
            pltpu.async_copy(g_hbm.at[src_v.at[cch + NBUF]], rows_v.at[b],
                             sems[b])
        return carry

    lax.fori_loop(0, CPT // NBUF - 1, group, 0)
    for b in range(NBUF):
        cch = CPT - NBUF + b
        pltpu.make_async_copy(g_hbm.at[src_v.at[cch]], rows_v.at[b],
                              sems[b]).wait()
        pltpu.sync_copy(rows_v.at[b], acc_sh.at[dst_v.at[cch]], add=True)

    plsc.subcore_barrier()
    pltpu.sync_copy(acc_sh.at[pl.ds(lo, RPT)],
                    out_hbm.at[pl.ds(c * NPAD + lo, RPT)])


def _make_prop(F):
    return functools.partial(
        pl.kernel,
        out_type=jax.ShapeDtypeStruct((NC * NPAD, F), jnp.float32),
        mesh=_sc_mesh(),
        compiler_params=_SC_PARAMS,
        scratch_types=[
            pltpu.VMEM((CPT, CH), jnp.int32),
            pltpu.VMEM((CPT, CH), jnp.int32),
            pltpu.VMEM((NBUF, CH, F), jnp.float32),
            pltpu.VMEM_SHARED((NPAD, F), jnp.float32),
        ] + [pltpu.SemaphoreType.DMA] * NBUF,
    )(_prop_body)


_prop8 = _make_prop(8)


BLK = 2000


def _t1_body(dg0_ref, dg1_ref, x_ref, w_ref, dinv_ref, g_ref):
    deg = dg0_ref[...] + dg1_ref[...] + 1.0
    dinv = 1.0 / jnp.sqrt(deg)
    dinv_ref[...] = dinv
    h = jnp.dot(x_ref[...], w_ref[...], preferred_element_type=jnp.float32)
    g_ref[...] = dinv * h


def _t1(dg0, dg1, x, W1):
    return pl.pallas_call(
        _t1_body,
        grid=(N // BLK,),
        in_specs=[
            pl.BlockSpec((BLK, 1), lambda i: (i, 0)),
            pl.BlockSpec((BLK, 1), lambda i: (i, 0)),
            pl.BlockSpec((BLK, D), lambda i: (i, 0)),
            pl.BlockSpec((D, 8), lambda i: (0, 0)),
        ],
        out_specs=[
            pl.BlockSpec((BLK, 1), lambda i: (i, 0)),
            pl.BlockSpec((BLK, 8), lambda i: (i, 0)),
        ],
        out_shape=[
            jax.ShapeDtypeStruct((N, 1), jnp.float32),
            jax.ShapeDtypeStruct((N, 8), jnp.float32),
        ],
    )(dg0, dg1, x, W1)


def _elu(v):
    return jnp.where(v > 0, v, jnp.exp(v) - 1.0)


def _rowspecs(n, F):
    return [pl.BlockSpec((BLK, F), lambda i: (i, 0)) for _ in range(n)]


def _tpost_body(a0_ref, a1_ref, g_ref, dinv_ref, b_ref, o_ref):
    dinv = dinv_ref[...]
    act = dinv * (a0_ref[...] + a1_ref[...] + g_ref[...]) + b_ref[...]
    o_ref[...] = dinv * _elu(act)


def _tpost(a0, a1, g, dinv, b):
    F = g.shape[1]
    return pl.pallas_call(
        _tpost_body,
        grid=(N // BLK,),
        in_specs=_rowspecs(3, F) + [
            pl.BlockSpec((BLK, 1), lambda i: (i, 0)),
            pl.BlockSpec((1, F), lambda i: (0, 0)),
        ],
        out_specs=pl.BlockSpec((BLK, F), lambda i: (i, 0)),
        out_shape=jax.ShapeDtypeStruct((N, F), jnp.float32),
    )(a0, a1, g, dinv, b)


def _tmm2_body(a0_ref, a1_ref, u_ref, dinv_ref, b_ref, w2_ref, w3_ref, o_ref):
    dinv = dinv_ref[...]
    t = dinv * (a0_ref[...] + a1_ref[...] + u_ref[...])
    h = _elu(jnp.dot(t, w2_ref[...], preferred_element_type=jnp.float32)
             + b_ref[...])
    o_ref[...] = dinv * jnp.dot(h, w3_ref[...],
                                preferred_element_type=jnp.float32)


def _tmm2(a0, a1, u, dinv, b, W2, W3):
    Fi, Fm = W2.shape
    Fo = W3.shape[1]
    return pl.pallas_call(
        _tmm2_body,
        grid=(N // BLK,),
        in_specs=_rowspecs(3, Fi) + [
            pl.BlockSpec((BLK, 1), lambda i: (i, 0)),
            pl.BlockSpec((1, Fm), lambda i: (0, 0)),
            pl.BlockSpec((Fi, Fm), lambda i: (0, 0)),
            pl.BlockSpec((Fm, Fo), lambda i: (0, 0)),
        ],
        out_specs=pl.BlockSpec((BLK, Fo), lambda i: (i, 0)),
        out_shape=jax.ShapeDtypeStruct((N, Fo), jnp.float32),
    )(a0, a1, u, dinv, b, W2, W3)


def _tfin_body(a0_ref, a1_ref, u_ref, dinv_ref, b_ref, w_ref, o_ref):
    t = dinv_ref[...] * (a0_ref[...] + a1_ref[...] + u_ref[...])
    act = jnp.dot(t, w_ref[...], preferred_element_type=jnp.float32) + b_ref[...]
    m = jnp.max(act, axis=1, keepdims=True)
    sft = act - m
    o_ref[...] = sft - jnp.log(jnp.sum(jnp.exp(sft), axis=1, keepdims=True))


def _tfin(a0, a1, u, dinv, b, W):
    Fi, Fo = W.shape
    return pl.pallas_call(
        _tfin_body,
        grid=(N // BLK,),
        in_specs=_rowspecs(3, Fi) + [
            pl.BlockSpec((BLK, 1), lambda i: (i, 0)),
            pl.BlockSpec((1, Fo), lambda i: (0, 0)),
            pl.BlockSpec((Fi, Fo), lambda i: (0, 0)),
        ],
        out_specs=pl.BlockSpec((BLK, Fo), lambda i: (i, 0)),
        out_shape=jax.ShapeDtypeStruct((N, Fo), jnp.float32),
    )(a0, a1, u, dinv, b, W)


def kernel(x, edge_index, laplacian_index, laplacian_weight,
           W1, b1, W2, b2, W3, b3, W4, b4):
    del laplacian_index, laplacian_weight
    src = edge_index[0]
    dst = edge_index[1]
    npad = EPAD - E
    src_r = jnp.concatenate([src, jnp.zeros((npad,), jnp.int32)]
                            ).reshape(NW, CPT, CH)
    dst_r = jnp.concatenate([dst, jnp.full((npad,), DUMMY, jnp.int32)]
                            ).reshape(NW, CPT, CH)

    zeros1 = jnp.zeros((NPAD,), jnp.float32)
    zeros8 = jnp.zeros((NPAD, 8), jnp.float32)
    deg2 = _deg_kernel(dst_r, zeros1)
    dg0 = deg2[:N, None]
    dg1 = deg2[NPAD:NPAD + N, None]

    dinv, g1 = _t1(dg0, dg1, x, W1)

    acc = _prop8(g1, src_r, dst_r, zeros8)
    u2 = _tpost(acc[:N], acc[NPAD:NPAD + N], g1, dinv, b1.reshape(1, 8))

    acc = _prop8(u2, src_r, dst_r, zeros8)
    g3 = _tmm2(acc[:N], acc[NPAD:NPAD + N], u2, dinv, b2.reshape(1, 16),
               W2, W3)

    acc = _prop8(g3, src_r, dst_r, zeros8)
    u4 = _tpost(acc[:N], acc[NPAD:NPAD + N], g3, dinv, b3.reshape(1, 8))

    acc = _prop8(u4, src_r, dst_r, zeros8)
    return _tfin(acc[:N], acc[NPAD:NPAD + N], u4, dinv, b4.reshape(1, 2), W4)


# speedup vs baseline: 51.9900x; 1.4351x over previous
import functools

import numpy as np
import jax
import jax.numpy as jnp
from jax import lax
from jax.experimental import pallas as pl
from jax.experimental.pallas import tpu as pltpu
from jax.experimental.pallas import tpu_sc as plsc

N = 10000
E = 320000
D = 128

NC, NS, L = 2, 16, 16
NW = NC * NS
CH = 80
CPT = 125
NPAD = 10240
RPT = NPAD // NS
NBUF = 5


def _sc_mesh():
    return plsc.VectorSubcoreMesh(core_axis_name="c", subcore_axis_name="s",
                                  num_cores=NC, num_subcores=NS)


_SC_PARAMS = pltpu.CompilerParams(use_tc_tiling_on_sc=False)


def _deg_body(ei_hbm, zeros_hbm, out_hbm, dst_v, ones_v, acc_sh):
    c = lax.axis_index("c")
    s = lax.axis_index("s")
    wid = c * NS + s
    lo = s * RPT
    pltpu.sync_copy(zeros_hbm.at[pl.ds(lo, RPT)], acc_sh.at[pl.ds(lo, RPT)])
    pltpu.sync_copy(ei_hbm.at[1, wid], dst_v)
    for i in range(CH // L):
        ones_v[pl.ds(i * L, L)] = jnp.full((L,), 1.0, jnp.float32)
    plsc.subcore_barrier()

    def chunk(j, carry):
        pltpu.sync_copy(ones_v, acc_sh.at[dst_v.at[j]], add=True)
        return carry

    lax.fori_loop(0, CPT, chunk, 0)
    plsc.subcore_barrier()
    pltpu.sync_copy(acc_sh.at[pl.ds(lo, RPT)],
                    out_hbm.at[pl.ds(c * NPAD + lo, RPT)])


_deg_kernel = functools.partial(
    pl.kernel,
    out_type=jax.ShapeDtypeStruct((NC * NPAD,), jnp.float32),
    mesh=_sc_mesh(),
    compiler_params=_SC_PARAMS,
    scratch_types=[
        pltpu.VMEM((CPT, CH), jnp.int32),
        pltpu.VMEM((CH,), jnp.float32),
        pltpu.VMEM_SHARED((NPAD,), jnp.float32),
    ],
)(_deg_body)


def _prop_body(g_hbm, ei_hbm, zeros_hbm, out_hbm,
               src_v, dst_v, rows_v, acc_sh, *sems):
    c = lax.axis_index("c")
    s = lax.axis_index("s")
    wid = c * NS + s
    lo = s * RPT
    pltpu.async_copy(zeros_hbm.at[pl.ds(lo, RPT)], acc_sh.at[pl.ds(lo, RPT)],
                     sems[0])
    pltpu.async_copy(ei_hbm.at[0, wid], src_v, sems[1])
    pltpu.async_copy(ei_hbm.at[1, wid], dst_v, sems[2])
    pltpu.make_async_copy(zeros_hbm.at[pl.ds(lo, RPT)],
                          acc_sh.at[pl.ds(lo, RPT)], sems[0]).wait()
    pltpu.make_async_copy(ei_hbm.at[0, wid], src_v, sems[1]).wait()
    pltpu.make_async_copy(ei_hbm.at[1, wid], dst_v, sems[2]).wait()
    plsc.subcore_barrier()

    for b in range(NBUF):
        pltpu.async_copy(g_hbm.at[src_v.at[b]], rows_v.at[b], sems[b])

    def group(i, carry):
        base = i * NBUF
        for b in range(NBUF):
            cch = base + b
            pltpu.make_async_copy(g_hbm.at[src_v.at[cch]], rows_v.at[b],
                                  sems[b]).wait()
            pltpu.sync_copy(rows_v.at[b], acc_sh.at[dst_v.at[cch]], add=True)
            pltpu.async_copy(g_hbm.at[src_v.at[cch + NBUF]], rows_v.at[b],
                             sems[b])
        return carry

    lax.fori_loop(0, CPT // NBUF - 1, group, 0)
    for b in range(NBUF):
        cch = CPT - NBUF + b
        pltpu.make_async_copy(g_hbm.at[src_v.at[cch]], rows_v.at[b],
                              sems[b]).wait()
        pltpu.sync_copy(rows_v.at[b], acc_sh.at[dst_v.at[cch]], add=True)

    plsc.subcore_barrier()
    pltpu.sync_copy(acc_sh.at[pl.ds(lo, RPT)],
                    out_hbm.at[pl.ds(c * NPAD + lo, RPT)])


_prop8 = functools.partial(
    pl.kernel,
    out_type=jax.ShapeDtypeStruct((NC * NPAD, 8), jnp.float32),
    mesh=_sc_mesh(),
    compiler_params=_SC_PARAMS,
    scratch_types=[
        pltpu.VMEM((CPT, CH), jnp.int32),
        pltpu.VMEM((CPT, CH), jnp.int32),
        pltpu.VMEM((NBUF, CH, 8), jnp.float32),
        pltpu.VMEM_SHARED((NPAD, 8), jnp.float32),
    ] + [pltpu.SemaphoreType.DMA] * NBUF,
)(_prop_body)


def _elu(v):
    return jnp.where(v > 0, v, jnp.exp(v) - 1.0)


def _halves(acc_ref):
    return acc_ref[pl.ds(0, NPAD), :] + acc_ref[pl.ds(NPAD, NPAD), :]


def _full(F):
    return pl.BlockSpec((NPAD, F), lambda i: (0, 0))


def _two(F):
    return pl.BlockSpec((2 * NPAD, F), lambda i: (0, 0))


def _t1_body(dg_ref, x_ref, w_ref, dinv_ref, g_ref):
    deg = _halves(dg_ref) + 1.0
    dinv = 1.0 / jnp.sqrt(deg)
    dinv_ref[...] = dinv
    h = jnp.dot(x_ref[...], w_ref[...], preferred_element_type=jnp.float32)
    g_ref[...] = dinv * h


def _t1(dg, x, W1):
    return pl.pallas_call(
        _t1_body,
        grid=(1,),
        in_specs=[_two(1), _full(D), pl.BlockSpec((D, 8), lambda i: (0, 0))],
        out_specs=[_full(1), _full(8)],
        out_shape=[
            jax.ShapeDtypeStruct((NPAD, 1), jnp.float32),
            jax.ShapeDtypeStruct((NPAD, 8), jnp.float32),
        ],
    )(dg, x, W1)


def _tpost_body(acc_ref, g_ref, dinv_ref, b_ref, o_ref):
    dinv = dinv_ref[...]
    act = dinv * (_halves(acc_ref) + g_ref[...]) + b_ref[...]
    o_ref[...] = dinv * _elu(act)


def _tpost(acc, g, dinv, b):
    return pl.pallas_call(
        _tpost_body,
        grid=(1,),
        in_specs=[_two(8), _full(8), _full(1),
                  pl.BlockSpec((1, 8), lambda i: (0, 0))],
        out_specs=_full(8),
        out_shape=jax.ShapeDtypeStruct((NPAD, 8), jnp.float32),
    )(acc, g, dinv, b)


def _tmm2_body(acc_ref, u_ref, dinv_ref, b_ref, w2_ref, w3_ref, o_ref):
    dinv = dinv_ref[...]
    t = dinv * (_halves(acc_ref) + u_ref[...])
    h = _elu(jnp.dot(t, w2_ref[...], preferred_element_type=jnp.float32)
             + b_ref[...])
    o_ref[...] = dinv * jnp.dot(h, w3_ref[...],
                                preferred_element_type=jnp.float32)


def _tmm2(acc, u, dinv, b, W2, W3):
    return pl.pallas_call(
        _tmm2_body,
        grid=(1,),
        in_specs=[_two(8), _full(8), _full(1),
                  pl.BlockSpec((1, 16), lambda i: (0, 0)),
                  pl.BlockSpec((8, 16), lambda i: (0, 0)),
                  pl.BlockSpec((16, 8), lambda i: (0, 0))],
        out_specs=_full(8),
        out_shape=jax.ShapeDtypeStruct((NPAD, 8), jnp.float32),
    )(acc, u, dinv, b, W2, W3)


def _tfin_body(acc_ref, u_ref, dinv_ref, b_ref, w_ref, o_ref):
    t = dinv_ref[...] * (_halves(acc_ref) + u_ref[...])
    act = jnp.dot(t, w_ref[...], preferred_element_type=jnp.float32) + b_ref[...]
    m = jnp.max(act, axis=1, keepdims=True)
    sft = act - m
    o_ref[...] = sft - jnp.log(jnp.sum(jnp.exp(sft), axis=1, keepdims=True))


def _tfin(acc, u, dinv, b, W):
    return pl.pallas_call(
        _tfin_body,
        grid=(1,),
        in_specs=[_two(8), _full(8), _full(1),
                  pl.BlockSpec((1, 2), lambda i: (0, 0)),
                  pl.BlockSpec((8, 2), lambda i: (0, 0))],
        out_specs=pl.BlockSpec((NPAD, 2), lambda i: (0, 0)),
        out_shape=jax.ShapeDtypeStruct((N, 2), jnp.float32),
    )(acc, u, dinv, b, W)


_ZEROS1 = np.zeros((NPAD,), np.float32)
_ZEROS8 = np.zeros((NPAD, 8), np.float32)


def kernel(x, edge_index, laplacian_index, laplacian_weight,
           W1, b1, W2, b2, W3, b3, W4, b4):
    del laplacian_index, laplacian_weight
    er = edge_index.reshape(2, NW, CPT, CH)

    deg2 = _deg_kernel(er, _ZEROS1)
    dinv, g1 = _t1(deg2.reshape(2 * NPAD, 1), x, W1)

    acc = _prop8(g1, er, _ZEROS8)
    u2 = _tpost(acc.reshape(2 * NPAD, 8), g1, dinv, b1.reshape(1, 8))

    acc = _prop8(u2, er, _ZEROS8)
    g3 = _tmm2(acc.reshape(2 * NPAD, 8), u2, dinv, b2.reshape(1, 16), W2, W3)

    acc = _prop8(g3, er, _ZEROS8)
    u4 = _tpost(acc.reshape(2 * NPAD, 8), g3, dinv, b3.reshape(1, 8))

    acc = _prop8(u4, er, _ZEROS8)
    return _tfin(acc.reshape(2 * NPAD, 8), u4, dinv, b4.reshape(1, 2), W4)


# speedup vs baseline: 61.4141x; 1.1813x over previous
import functools

import numpy as np
import jax
import jax.numpy as jnp
from jax import lax
from jax.experimental import pallas as pl
from jax.experimental.pallas import tpu as pltpu
from jax.experimental.pallas import tpu_sc as plsc

N = 10000
E = 320000
D = 128

NC, NS, L = 2, 16, 16
NW = NC * NS
CH = 400
CPT = 25
NPAD = 10240
RPT = NPAD // NS
NBUF = 5


def _sc_mesh():
    return plsc.VectorSubcoreMesh(core_axis_name="c", subcore_axis_name="s",
                                  num_cores=NC, num_subcores=NS)


_SC_PARAMS = pltpu.CompilerParams(use_tc_tiling_on_sc=False)


def _deg_body(ei_hbm, zeros_hbm, out_hbm, dst_v, ones_v, acc_sh):
    c = lax.axis_index("c")
    s = lax.axis_index("s")
    wid = c * NS + s
    lo = s * RPT
    pltpu.sync_copy(zeros_hbm.at[pl.ds(lo, RPT)], acc_sh.at[pl.ds(lo, RPT)])
    pltpu.sync_copy(ei_hbm.at[1, wid], dst_v)
    for i in range(CH // L):
        ones_v[pl.ds(i * L, L)] = jnp.full((L,), 1.0, jnp.float32)
    plsc.subcore_barrier()

    def chunk(j, carry):
        pltpu.sync_copy(ones_v, acc_sh.at[dst_v.at[j]], add=True)
        return carry

    lax.fori_loop(0, CPT, chunk, 0)
    plsc.subcore_barrier()
    pltpu.sync_copy(acc_sh.at[pl.ds(lo, RPT)],
                    out_hbm.at[pl.ds(c * NPAD + lo, RPT)])


_deg_kernel = functools.partial(
    pl.kernel,
    out_type=jax.ShapeDtypeStruct((NC * NPAD,), jnp.float32),
    mesh=_sc_mesh(),
    compiler_params=_SC_PARAMS,
    scratch_types=[
        pltpu.VMEM((CPT, CH), jnp.int32),
        pltpu.VMEM((CH,), jnp.float32),
        pltpu.VMEM_SHARED((NPAD,), jnp.float32),
    ],
)(_deg_body)


def _prop_body(g_hbm, ei_hbm, zeros_hbm, out_hbm,
               src_v, dst_v, rows_v, acc_sh, *sems):
    c = lax.axis_index("c")
    s = lax.axis_index("s")
    wid = c * NS + s
    lo = s * RPT
    pltpu.async_copy(zeros_hbm.at[pl.ds(lo, RPT)], acc_sh.at[pl.ds(lo, RPT)],
                     sems[0])
    pltpu.async_copy(ei_hbm.at[0, wid], src_v, sems[1])
    pltpu.async_copy(ei_hbm.at[1, wid], dst_v, sems[2])
    pltpu.make_async_copy(zeros_hbm.at[pl.ds(lo, RPT)],
                          acc_sh.at[pl.ds(lo, RPT)], sems[0]).wait()
    pltpu.make_async_copy(ei_hbm.at[0, wid], src_v, sems[1]).wait()
    pltpu.make_async_copy(ei_hbm.at[1, wid], dst_v, sems[2]).wait()
    plsc.subcore_barrier()

    for b in range(NBUF):
        pltpu.async_copy(g_hbm.at[src_v.at[b]], rows_v.at[b], sems[b])

    def group(i, carry):
        base = i * NBUF
        for b in range(NBUF):
            cch = base + b
            pltpu.make_async_copy(g_hbm.at[src_v.at[cch]], rows_v.at[b],
                                  sems[b]).wait()
            pltpu.sync_copy(rows_v.at[b], acc_sh.at[dst_v.at[cch]], add=True)
            pltpu.async_copy(g_hbm.at[src_v.at[cch + NBUF]], rows_v.at[b],
                             sems[b])
        return carry

    lax.fori_loop(0, CPT // NBUF - 1, group, 0)
    for b in range(NBUF):
        cch = CPT - NBUF + b
        pltpu.make_async_copy(g_hbm.at[src_v.at[cch]], rows_v.at[b],
                              sems[b]).wait()
        pltpu.sync_copy(rows_v.at[b], acc_sh.at[dst_v.at[cch]], add=True)

    plsc.subcore_barrier()
    pltpu.sync_copy(acc_sh.at[pl.ds(lo, RPT)],
                    out_hbm.at[pl.ds(c * NPAD + lo, RPT)])


_prop8 = functools.partial(
    pl.kernel,
    out_type=jax.ShapeDtypeStruct((NC * NPAD, 8), jnp.float32),
    mesh=_sc_mesh(),
    compiler_params=_SC_PARAMS,
    scratch_types=[
        pltpu.VMEM((CPT, CH), jnp.int32),
        pltpu.VMEM((CPT, CH), jnp.int32),
        pltpu.VMEM((NBUF, CH, 8), jnp.float32),
        pltpu.VMEM_SHARED((NPAD, 8), jnp.float32),
    ] + [pltpu.SemaphoreType.DMA] * NBUF,
)(_prop_body)


def _elu(v):
    return jnp.where(v > 0, v, jnp.exp(v) - 1.0)


def _halves(acc_ref):
    return acc_ref[pl.ds(0, NPAD), :] + acc_ref[pl.ds(NPAD, NPAD), :]


def _full(F):
    return pl.BlockSpec((NPAD, F), lambda i: (0, 0))


def _two(F):
    return pl.BlockSpec((2 * NPAD, F), lambda i: (0, 0))


def _t1_body(dg_ref, x_ref, w_ref, dinv_ref, g_ref):
    deg = _halves(dg_ref) + 1.0
    dinv = 1.0 / jnp.sqrt(deg)
    dinv_ref[...] = dinv
    h = jnp.dot(x_ref[...], w_ref[...], preferred_element_type=jnp.float32)
    g_ref[...] = dinv * h


def _t1(dg, x, W1):
    return pl.pallas_call(
        _t1_body,
        grid=(1,),
        in_specs=[_two(1), _full(D), pl.BlockSpec((D, 8), lambda i: (0, 0))],
        out_specs=[_full(1), _full(8)],
        out_shape=[
            jax.ShapeDtypeStruct((NPAD, 1), jnp.float32),
            jax.ShapeDtypeStruct((NPAD, 8), jnp.float32),
        ],
    )(dg, x, W1)


def _tpost_body(acc_ref, g_ref, dinv_ref, b_ref, o_ref):
    dinv = dinv_ref[...]
    act = dinv * (_halves(acc_ref) + g_ref[...]) + b_ref[...]
    o_ref[...] = dinv * _elu(act)


def _tpost(acc, g, dinv, b):
    return pl.pallas_call(
        _tpost_body,
        grid=(1,),
        in_specs=[_two(8), _full(8), _full(1),
                  pl.BlockSpec((1, 8), lambda i: (0, 0))],
        out_specs=_full(8),
        out_shape=jax.ShapeDtypeStruct((NPAD, 8), jnp.float32),
    )(acc, g, dinv, b)


def _tmm2_body(acc_ref, u_ref, dinv_ref, b_ref, w2_ref, w3_ref, o_ref):
    dinv = dinv_ref[...]
    t = dinv * (_halves(acc_ref) + u_ref[...])
    h = _elu(jnp.dot(t, w2_ref[...], preferred_element_type=jnp.float32)
             + b_ref[...])
    o_ref[...] = dinv * jnp.dot(h, w3_ref[...],
                                preferred_element_type=jnp.float32)


def _tmm2(acc, u, dinv, b, W2, W3):
    return pl.pallas_call(
        _tmm2_body,
        grid=(1,),
        in_specs=[_two(8), _full(8), _full(1),
                  pl.BlockSpec((1, 16), lambda i: (0, 0)),
                  pl.BlockSpec((8, 16), lambda i: (0, 0)),
                  pl.BlockSpec((16, 8), lambda i: (0, 0))],
        out_specs=_full(8),
        out_shape=jax.ShapeDtypeStruct((NPAD, 8), jnp.float32),
    )(acc, u, dinv, b, W2, W3)


def _tfin_body(acc_ref, u_ref, dinv_ref, b_ref, w_ref, o_ref):
    t = dinv_ref[...] * (_halves(acc_ref) + u_ref[...])
    act = jnp.dot(t, w_ref[...], preferred_element_type=jnp.float32) + b_ref[...]
    m = jnp.max(act, axis=1, keepdims=True)
    sft = act - m
    o_ref[...] = sft - jnp.log(jnp.sum(jnp.exp(sft), axis=1, keepdims=True))


def _tfin(acc, u, dinv, b, W):
    return pl.pallas_call(
        _tfin_body,
        grid=(1,),
        in_specs=[_two(8), _full(8), _full(1),
                  pl.BlockSpec((1, 2), lambda i: (0, 0)),
                  pl.BlockSpec((8, 2), lambda i: (0, 0))],
        out_specs=pl.BlockSpec((NPAD, 2), lambda i: (0, 0)),
        out_shape=jax.ShapeDtypeStruct((N, 2), jnp.float32),
    )(acc, u, dinv, b, W)


_ZEROS1 = np.zeros((NPAD,), np.float32)
_ZEROS8 = np.zeros((NPAD, 8), np.float32)


def kernel(x, edge_index, laplacian_index, laplacian_weight,
           W1, b1, W2, b2, W3, b3, W4, b4):
    del laplacian_index, laplacian_weight
    er = edge_index.reshape(2, NW, CPT, CH)

    deg2 = _deg_kernel(er, _ZEROS1)
    dinv, g1 = _t1(deg2.reshape(2 * NPAD, 1), x, W1)

    acc = _prop8(g1, er, _ZEROS8)
    u2 = _tpost(acc.reshape(2 * NPAD, 8), g1, dinv, b1.reshape(1, 8))

    acc = _prop8(u2, er, _ZEROS8)
    g3 = _tmm2(acc.reshape(2 * NPAD, 8), u2, dinv, b2.reshape(1, 16), W2, W3)

    acc = _prop8(g3, er, _ZEROS8)
    u4 = _tpost(acc.reshape(2 * NPAD, 8), g3, dinv, b3.reshape(1, 8))

    acc = _prop8(u4, er, _ZEROS8)
    return _tfin(acc.reshape(2 * NPAD, 8), u4, dinv, b4.reshape(1, 2), W4)
